# baseline (speedup 1.0000x reference)
import jax
import jax.numpy as jnp
from jax.experimental import pallas as pl

_BN = 512
_NC = 8


def _mm_kernel(x0_ref, x1_ref, x2_ref, x3_ref, x4_ref, x5_ref, x6_ref, x7_ref, w_ref, o_ref):
    wb = w_ref[...].astype(jnp.bfloat16)
    cm = x0_ref.shape[0]
    for c, xc in enumerate((x0_ref, x1_ref, x2_ref, x3_ref, x4_ref, x5_ref, x6_ref, x7_ref)):
        o_ref[c * cm:(c + 1) * cm, :] = jax.lax.dot_general(
            xc[...].astype(jnp.bfloat16), wb, (((1,), (1,)), ((), ())),
            preferred_element_type=jnp.float32)


def kernel(x, weight):
    M, K = x.shape
    N, _ = weight.shape
    cm = M // _NC
    x_specs = [
        pl.BlockSpec((cm, K), (lambda j, c=c: (c, 0))) for c in range(_NC)
    ]
    return pl.pallas_call(
        _mm_kernel,
        grid=(N // _BN,),
        in_specs=x_specs + [pl.BlockSpec((_BN, K), lambda j: (j, 0))],
        out_specs=pl.BlockSpec((M, _BN), lambda j: (0, j)),
        out_shape=jax.ShapeDtypeStruct((M, N), jnp.float32),
    )(x, x, x, x, x, x, x, x, weight)
